# Initial kernel scaffold; baseline (speedup 1.0000x reference)
import jax
import jax.numpy as jnp
from jax.experimental import pallas as pl


def kernel(q, k, q_prune, k_prune, v, prev_attn_top_k_idx):
    raise NotImplementedError("write your pallas kernel here")


# speedup vs baseline: 11.8310x; 11.8310x over previous
import functools

import jax
import jax.numpy as jnp
from jax import lax
from jax.experimental import pallas as pl
from jax.experimental.pallas import tpu as pltpu
from jax.experimental.pallas import tpu_sc as plsc

B, C, CP, CV, H, W, NK = 8, 192, 64, 192, 32, 32, 8
N = H * W
K5 = NK * 5
W48 = 48
ROWS = B * N
NTILES = 32
RPT = ROWS // NTILES
CH = 16
NCH = RPT // CH
RB = 512
BM = 256


def _k1_body(q_ref, k_ref, qp_ref, kp_ref, a_ref, p_ref):
    a_ref[0] = jnp.dot(q_ref[0], k_ref[0], preferred_element_type=jnp.float32)
    p_ref[0] = jnp.dot(qp_ref[0], kp_ref[0], preferred_element_type=jnp.float32)


def _k1(q_win, k_r, qp_win, kp_r):
    return pl.pallas_call(
        _k1_body,
        grid=(B, N // BM),
        in_specs=[
            pl.BlockSpec((1, BM, C), lambda b, i: (b, i, 0)),
            pl.BlockSpec((1, C, N), lambda b, i: (b, 0, 0)),
            pl.BlockSpec((1, BM, CP), lambda b, i: (b, i, 0)),
            pl.BlockSpec((1, CP, N), lambda b, i: (b, 0, 0)),
        ],
        out_specs=[
            pl.BlockSpec((1, BM, N), lambda b, i: (b, i, 0)),
            pl.BlockSpec((1, BM, N), lambda b, i: (b, i, 0)),
        ],
        out_shape=[
            jax.ShapeDtypeStruct((B, N, N), jnp.float32),
            jax.ShapeDtypeStruct((B, N, N), jnp.float32),
        ],
    )(q_win, k_r, qp_win, kp_r)


def _k2_body(prev_ref, a_hbm, p_hbm, idx_out, s_out, p_out,
             idx_vmem, a_chunk, p_chunk, idxbuf, sbuf, pbuf):
    wid = lax.axis_index("c") * 16 + lax.axis_index("s")
    tile_base = wid * RPT
    batch = wid // (N // RPT)
    n0 = (wid % (N // RPT)) * RPT

    pltpu.sync_copy(prev_ref.at[pl.ds(batch * N * NK, N * NK)], idx_vmem)

    iota16 = lax.broadcasted_iota(jnp.int32, (16,), 0)

    def chunk_body(ci, carry):
        row0 = tile_base + ci * CH
        pltpu.sync_copy(a_hbm.at[pl.ds(row0 * N, CH * N)], a_chunk)
        pltpu.sync_copy(p_hbm.at[pl.ds(row0 * N, CH * N)], p_chunk)

        n_vec = n0 + ci * CH + iota16
        y = lax.shift_right_logical(n_vec, 5)
        x = lax.bitwise_and(n_vec, 31)
        srcs = [
            n_vec,
            jnp.maximum(y - 1, 0) * 32 + x,
            jnp.minimum(y + 1, 31) * 32 + x,
            y * 32 + jnp.maximum(x - 1, 0),
            y * 32 + jnp.minimum(x + 1, 31),
        ]
        col0 = ci * CH
        for c in range(K5):
            g, t = c // 8, c % 8
            p_val = plsc.load_gather(idx_vmem, [srcs[g] * NK + t])
            py = lax.shift_right_logical(p_val, 5)
            px = lax.bitwise_and(p_val, 31)
            if g == 0:
                oidx = p_val
            elif g == 1:
                oidx = jnp.minimum(py + 1, 31) * 32 + px
            elif g == 2:
                oidx = jnp.maximum(py - 1, 0) * 32 + px
            elif g == 3:
                oidx = py * 32 + jnp.minimum(px + 1, 31)
            else:
                oidx = py * 32 + jnp.maximum(px - 1, 0)
            idxbuf[c, 0, pl.ds(col0, 16)] = oidx
            addr = iota16 * N + oidx
            sbuf[c, 0, pl.ds(col0, 16)] = plsc.load_gather(a_chunk, [addr])
            pbuf[c, 0, pl.ds(col0, 16)] = plsc.load_gather(p_chunk, [addr])
        return carry

    lax.fori_loop(0, NCH, chunk_body, 0)
    zero16 = jnp.zeros((16,), jnp.int32)
    zf16 = jnp.zeros((16,), jnp.float32)
    for c in range(K5, W48):
        for j in range(RPT // 16):
            idxbuf[c, 0, pl.ds(j * 16, 16)] = zero16
            sbuf[c, 0, pl.ds(j * 16, 16)] = zf16
            pbuf[c, 0, pl.ds(j * 16, 16)] = zf16

    pltpu.sync_copy(idxbuf, idx_out.at[:, pl.ds(wid, 1)])
    pltpu.sync_copy(sbuf, s_out.at[:, pl.ds(wid, 1)])
    pltpu.sync_copy(pbuf, p_out.at[:, pl.ds(wid, 1)])


def _k2(prev_flat, a_flat, p_flat):
    mesh = plsc.VectorSubcoreMesh(core_axis_name="c", subcore_axis_name="s")
    return pl.kernel(
        _k2_body,
        out_type=[
            jax.ShapeDtypeStruct((W48, NTILES, RPT), jnp.int32),
            jax.ShapeDtypeStruct((W48, NTILES, RPT), jnp.float32),
            jax.ShapeDtypeStruct((W48, NTILES, RPT), jnp.float32),
        ],
        mesh=mesh,
        compiler_params=pltpu.CompilerParams(needs_layout_passes=False),
        scratch_types=[
            pltpu.VMEM((N * NK,), jnp.int32),
            pltpu.VMEM((CH * N,), jnp.float32),
            pltpu.VMEM((CH * N,), jnp.float32),
            pltpu.VMEM((W48, 1, RPT), jnp.int32),
            pltpu.VMEM((W48, 1, RPT), jnp.float32),
            pltpu.VMEM((W48, 1, RPT), jnp.float32),
        ],
    )(prev_flat, a_flat, p_flat)


def _k3_body(s_ref, p_ref, i_ref, w_ref, scat_ref, topk_ref):
    s = s_ref[...]
    pv = p_ref[...]
    idx = i_ref[...]
    sub = lax.broadcasted_iota(jnp.int32, (W48, RB), 0)
    tmask = sub < K5

    smooth = jnp.float32(C ** 0.5)
    s_m = jnp.where(tmask, s / smooth, -jnp.inf)
    mx = jnp.max(s_m, axis=0, keepdims=True)
    e = jnp.exp(s_m - mx)
    denom = jnp.sum(e, axis=0, keepdims=True)
    attn = e / denom
    w = jnp.where(tmask & (pv > 0), attn, 0.0)
    conf = jnp.sum(w, axis=0, keepdims=True)

    accw = jnp.zeros((W48, RB), jnp.float32)
    minpos = jnp.full((W48, RB), W48, jnp.int32)
    for tp in range(K5):
        eq = idx == idx[tp:tp + 1, :]
        accw = accw + jnp.where(eq, w[tp:tp + 1, :], 0.0)
        minpos = jnp.minimum(minpos, jnp.where(eq, tp, W48))
    first = (minpos == sub) & tmask
    w_comb = jnp.where(first, accw, 0.0)
    scat_ref[...] = jnp.where(first, idx, -1)
    w_ref[...] = jnp.where(sub == (W48 - 1), conf, w_comb)

    vals = jnp.where(tmask, attn, -1.0)
    for i in range(NK):
        mxv = jnp.max(vals, axis=0, keepdims=True)
        pos = jnp.min(jnp.where(vals == mxv, sub, W48), axis=0, keepdims=True)
        sel = sub == pos
        topk_ref[i:i + 1, :] = jnp.sum(jnp.where(sel, idx, 0), axis=0,
                                       keepdims=True)
        vals = jnp.where(sel, -1.0, vals)


def _k3(s_t, p_t, idx_t):
    return pl.pallas_call(
        _k3_body,
        grid=(ROWS // RB,),
        in_specs=[
            pl.BlockSpec((W48, RB), lambda i: (0, i)),
            pl.BlockSpec((W48, RB), lambda i: (0, i)),
            pl.BlockSpec((W48, RB), lambda i: (0, i)),
        ],
        out_specs=[
            pl.BlockSpec((W48, RB), lambda i: (0, i)),
            pl.BlockSpec((W48, RB), lambda i: (0, i)),
            pl.BlockSpec((NK, RB), lambda i: (0, i)),
        ],
        out_shape=[
            jax.ShapeDtypeStruct((W48, ROWS), jnp.float32),
            jax.ShapeDtypeStruct((W48, ROWS), jnp.int32),
            jax.ShapeDtypeStruct((NK, ROWS), jnp.int32),
        ],
    )(s_t, p_t, idx_t)


def _k4_body(w_hbm, scat_hbm, s_flat_out, wbuf, scatbuf, s_chunk):
    wid = lax.axis_index("c") * 16 + lax.axis_index("s")
    tile_base = wid * RPT

    pltpu.sync_copy(w_hbm.at[:, pl.ds(wid, 1)], wbuf)
    pltpu.sync_copy(scat_hbm.at[:, pl.ds(wid, 1)], scatbuf)

    iota16 = lax.broadcasted_iota(jnp.int32, (16,), 0)
    zf16 = jnp.zeros((16,), jnp.float32)

    def zero_body(j, carry):
        s_chunk[pl.ds(j * 16, 16)] = zf16
        return carry

    lax.fori_loop(0, CH * N // 16, zero_body, 0)

    def chunk_body(ci, carry):
        col0 = ci * CH
        for c in range(K5):
            w_vec = wbuf[c, 0, pl.ds(col0, 16)]
            scat = scatbuf[c, 0, pl.ds(col0, 16)]
            addr = iota16 * N + jnp.maximum(scat, 0)
            plsc.store_scatter(s_chunk, [addr], w_vec, mask=scat >= 0)
        pltpu.sync_copy(s_chunk,
                        s_flat_out.at[pl.ds((tile_base + ci * CH) * N, CH * N)])
        for c in range(K5):
            scat = scatbuf[c, 0, pl.ds(col0, 16)]
            addr = iota16 * N + jnp.maximum(scat, 0)
            plsc.store_scatter(s_chunk, [addr], zf16, mask=scat >= 0)
        return carry

    lax.fori_loop(0, NCH, chunk_body, 0)


def _k4(w3, scat3):
    mesh = plsc.VectorSubcoreMesh(core_axis_name="c", subcore_axis_name="s")
    return pl.kernel(
        _k4_body,
        out_type=jax.ShapeDtypeStruct((ROWS * N,), jnp.float32),
        mesh=mesh,
        compiler_params=pltpu.CompilerParams(needs_layout_passes=False),
        scratch_types=[
            pltpu.VMEM((W48, 1, RPT), jnp.float32),
            pltpu.VMEM((W48, 1, RPT), jnp.int32),
            pltpu.VMEM((CH * N,), jnp.float32),
        ],
    )(w3, scat3)


def _k5_body(s_ref, v_ref, o_ref):
    o_ref[0] = jnp.dot(s_ref[0], v_ref[0], preferred_element_type=jnp.float32)


def _k5(s_dense, v_win):
    return pl.pallas_call(
        _k5_body,
        grid=(B, N // BM),
        in_specs=[
            pl.BlockSpec((1, BM, N), lambda b, i: (b, i, 0)),
            pl.BlockSpec((1, N, CV), lambda b, i: (b, 0, 0)),
        ],
        out_specs=pl.BlockSpec((1, BM, CV), lambda b, i: (b, i, 0)),
        out_shape=jax.ShapeDtypeStruct((B, N, CV), jnp.float32),
    )(s_dense, v_win)


@jax.jit
def kernel(q, k, q_prune, k_prune, v, prev_attn_top_k_idx):
    q_win = q.reshape(B, C, N).transpose(0, 2, 1)
    k_r = k.reshape(B, C, N)
    qp_win = q_prune.reshape(B, CP, N).transpose(0, 2, 1)
    kp_r = k_prune.reshape(B, CP, N)
    v_win = v.reshape(B, CV, N).transpose(0, 2, 1)

    a_d, p_d = _k1(q_win, k_r, qp_win, kp_r)

    idx3, s3, p3 = _k2(prev_attn_top_k_idx.reshape(ROWS * NK),
                       a_d.reshape(ROWS * N), p_d.reshape(ROWS * N))

    idx_t = idx3.reshape(W48, ROWS)
    w_t, scat_t, topk_t = _k3(s3.reshape(W48, ROWS), p3.reshape(W48, ROWS),
                              idx_t)

    s_flat = _k4(w_t.reshape(W48, NTILES, RPT),
                 scat_t.reshape(W48, NTILES, RPT))

    out_win = _k5(s_flat.reshape(B, N, N), v_win)

    output = out_win.transpose(0, 2, 1).reshape(B, CV, H, W)
    this_attn_top_k_idx = topk_t.T.reshape(B, N, NK)
    conf = w_t[W48 - 1].reshape(B, 1, H, W)
    return output, this_attn_top_k_idx, conf


# speedup vs baseline: 14.2319x; 1.2029x over previous
import jax
import jax.numpy as jnp
from jax import lax
from jax.experimental import pallas as pl
from jax.experimental.pallas import tpu as pltpu
from jax.experimental.pallas import tpu_sc as plsc

B, C, CP, CV, H, W, NK = 8, 192, 64, 192, 32, 32, 8
N = H * W
K5 = NK * 5
W48 = 48
ROWS = B * N
NTILES = 32
RPT = ROWS // NTILES
CH = 16
NCH = RPT // CH
RB = 512
BM = 256


def _k1_body(q_ref, k_ref, qp_ref, kp_ref, a_ref):
    dn = (((0,), (0,)), ((), ()))
    a = lax.dot_general(q_ref[0], k_ref[0], dn,
                        preferred_element_type=jnp.float32)
    p = lax.dot_general(qp_ref[0], kp_ref[0], dn,
                        preferred_element_type=jnp.float32)
    ai = lax.bitcast_convert_type(a, jnp.int32)
    enc = jnp.where(p > 0, ai | 1, ai & -2)
    a_ref[0] = lax.bitcast_convert_type(enc, jnp.float32)


def _k1(q_r, k_r, qp_r, kp_r):
    return pl.pallas_call(
        _k1_body,
        grid=(B, N // BM),
        in_specs=[
            pl.BlockSpec((1, C, BM), lambda b, i: (b, 0, i)),
            pl.BlockSpec((1, C, N), lambda b, i: (b, 0, 0)),
            pl.BlockSpec((1, CP, BM), lambda b, i: (b, 0, i)),
            pl.BlockSpec((1, CP, N), lambda b, i: (b, 0, 0)),
        ],
        out_specs=pl.BlockSpec((1, BM, N), lambda b, i: (b, i, 0)),
        out_shape=jax.ShapeDtypeStruct((B, N, N), jnp.float32),
        compiler_params=pltpu.CompilerParams(
            dimension_semantics=("parallel", "parallel")),
    )(q_r, k_r, qp_r, kp_r)


def _k2_body(prev_ref, a_hbm, idx_out, s_out,
             idx_vmem, a_buf0, a_buf1, idxbuf, sbuf, sem0, sem1):
    wid = lax.axis_index("c") * 16 + lax.axis_index("s")
    tile_base = wid * RPT
    batch = wid // (N // RPT)
    n0 = (wid % (N // RPT)) * RPT

    pltpu.sync_copy(prev_ref.at[pl.ds(batch * N * NK, N * NK)], idx_vmem)

    iota16 = lax.broadcasted_iota(jnp.int32, (16,), 0)

    def chunk_src(ci):
        return a_hbm.at[pl.ds((tile_base + ci * CH) * N, CH * N)]

    def process(ci, a_chunk):
        n_vec = n0 + ci * CH + iota16
        y = lax.shift_right_logical(n_vec, 5)
        x = lax.bitwise_and(n_vec, 31)
        srcs = [
            n_vec,
            jnp.maximum(y - 1, 0) * 32 + x,
            jnp.minimum(y + 1, 31) * 32 + x,
            y * 32 + jnp.maximum(x - 1, 0),
            y * 32 + jnp.minimum(x + 1, 31),
        ]
        col0 = ci * CH
        for c in range(K5):
            g, t = c // 8, c % 8
            p_val = plsc.load_gather(idx_vmem, [srcs[g] * NK + t])
            py = lax.shift_right_logical(p_val, 5)
            px = lax.bitwise_and(p_val, 31)
            if g == 0:
                oidx = p_val
            elif g == 1:
                oidx = jnp.minimum(py + 1, 31) * 32 + px
            elif g == 2:
                oidx = jnp.maximum(py - 1, 0) * 32 + px
            elif g == 3:
                oidx = py * 32 + jnp.minimum(px + 1, 31)
            else:
                oidx = py * 32 + jnp.maximum(px - 1, 0)
            idxbuf[c, 0, pl.ds(col0, 16)] = oidx
            addr = iota16 * N + oidx
            sbuf[c, 0, pl.ds(col0, 16)] = plsc.load_gather(a_chunk, [addr])

    pltpu.async_copy(chunk_src(0), a_buf0, sem0)
    pltpu.async_copy(chunk_src(1), a_buf1, sem1)

    def pair_body(i, carry):
        ci0 = 2 * i
        ci1 = 2 * i + 1
        pltpu.make_async_copy(chunk_src(ci0), a_buf0, sem0).wait()
        process(ci0, a_buf0)
        pltpu.async_copy(chunk_src(jnp.minimum(ci0 + 2, NCH - 1)), a_buf0,
                         sem0)
        pltpu.make_async_copy(chunk_src(ci1), a_buf1, sem1).wait()
        process(ci1, a_buf1)
        pltpu.async_copy(chunk_src(jnp.minimum(ci1 + 2, NCH - 1)), a_buf1,
                         sem1)
        return carry

    lax.fori_loop(0, NCH // 2, pair_body, 0)
    pltpu.make_async_copy(chunk_src(NCH - 1), a_buf0, sem0).wait()
    pltpu.make_async_copy(chunk_src(NCH - 1), a_buf1, sem1).wait()

    zero16 = jnp.zeros((16,), jnp.int32)
    zf16 = jnp.zeros((16,), jnp.float32)
    for c in range(K5, W48):
        for j in range(RPT // 16):
            idxbuf[c, 0, pl.ds(j * 16, 16)] = zero16
            sbuf[c, 0, pl.ds(j * 16, 16)] = zf16

    pltpu.sync_copy(idxbuf, idx_out.at[:, pl.ds(wid, 1)])
    pltpu.sync_copy(sbuf, s_out.at[:, pl.ds(wid, 1)])


def _k2(prev_flat, a_flat):
    mesh = plsc.VectorSubcoreMesh(core_axis_name="c", subcore_axis_name="s")
    return pl.kernel(
        _k2_body,
        out_type=[
            jax.ShapeDtypeStruct((W48, NTILES, RPT), jnp.int32),
            jax.ShapeDtypeStruct((W48, NTILES, RPT), jnp.float32),
        ],
        mesh=mesh,
        compiler_params=pltpu.CompilerParams(needs_layout_passes=False),
        scratch_types=[
            pltpu.VMEM((N * NK,), jnp.int32),
            pltpu.VMEM((CH * N,), jnp.float32),
            pltpu.VMEM((CH * N,), jnp.float32),
            pltpu.VMEM((W48, 1, RPT), jnp.int32),
            pltpu.VMEM((W48, 1, RPT), jnp.float32),
            pltpu.SemaphoreType.DMA,
            pltpu.SemaphoreType.DMA,
        ],
    )(prev_flat, a_flat)


def _k3_body(s_ref, i_ref, w_ref, scat_ref, topk_ref):
    s_enc = lax.bitcast_convert_type(s_ref[...], jnp.int32)
    pmask = (s_enc & 1) == 1
    s = lax.bitcast_convert_type(s_enc & -2, jnp.float32)
    idx = i_ref[...]
    sub = lax.broadcasted_iota(jnp.int32, (W48, RB), 0)
    tmask = sub < K5

    smooth = jnp.float32(C ** 0.5)
    s_m = jnp.where(tmask, s / smooth, -jnp.inf)
    mx = jnp.max(s_m, axis=0, keepdims=True)
    e = jnp.exp(s_m - mx)
    denom = jnp.sum(e, axis=0, keepdims=True)
    attn = e / denom
    w = jnp.where(tmask & pmask, attn, 0.0)
    conf = jnp.sum(w, axis=0, keepdims=True)

    accw = jnp.zeros((W48, RB), jnp.float32)
    minpos = jnp.full((W48, RB), W48, jnp.int32)
    for tp in range(K5):
        eq = idx == idx[tp:tp + 1, :]
        accw = accw + jnp.where(eq, w[tp:tp + 1, :], 0.0)
        minpos = jnp.minimum(minpos, jnp.where(eq, tp, W48))
    first = (minpos == sub) & tmask
    w_comb = jnp.where(first, accw, 0.0)
    scat_ref[...] = jnp.where(first, idx, -1)
    w_ref[...] = jnp.where(sub == (W48 - 1), conf, w_comb)

    vals = jnp.where(tmask, attn, -1.0)
    for i in range(NK):
        mxv = jnp.max(vals, axis=0, keepdims=True)
        pos = jnp.min(jnp.where(vals == mxv, sub, W48), axis=0, keepdims=True)
        sel = sub == pos
        topk_ref[i:i + 1, :] = jnp.sum(jnp.where(sel, idx, 0), axis=0,
                                       keepdims=True)
        vals = jnp.where(sel, -1.0, vals)


def _k3(s_t, idx_t):
    return pl.pallas_call(
        _k3_body,
        grid=(ROWS // RB,),
        in_specs=[
            pl.BlockSpec((W48, RB), lambda i: (0, i)),
            pl.BlockSpec((W48, RB), lambda i: (0, i)),
        ],
        out_specs=[
            pl.BlockSpec((W48, RB), lambda i: (0, i)),
            pl.BlockSpec((W48, RB), lambda i: (0, i)),
            pl.BlockSpec((NK, RB), lambda i: (0, i)),
        ],
        out_shape=[
            jax.ShapeDtypeStruct((W48, ROWS), jnp.float32),
            jax.ShapeDtypeStruct((W48, ROWS), jnp.int32),
            jax.ShapeDtypeStruct((NK, ROWS), jnp.int32),
        ],
        compiler_params=pltpu.CompilerParams(
            dimension_semantics=("parallel",)),
    )(s_t, idx_t)


def _k4_body(w_hbm, scat_hbm, s_flat_out, wbuf, scatbuf, s_chunk):
    wid = lax.axis_index("c") * 16 + lax.axis_index("s")
    tile_base = wid * RPT

    pltpu.sync_copy(w_hbm.at[:, pl.ds(wid, 1)], wbuf)
    pltpu.sync_copy(scat_hbm.at[:, pl.ds(wid, 1)], scatbuf)

    iota16 = lax.broadcasted_iota(jnp.int32, (16,), 0)
    zf16 = jnp.zeros((16,), jnp.float32)

    def zero_body(j, carry):
        s_chunk[pl.ds(j * 16, 16)] = zf16
        return carry

    lax.fori_loop(0, CH * N // 16, zero_body, 0)

    def chunk_body(ci, carry):
        col0 = ci * CH
        for c in range(K5):
            w_vec = wbuf[c, 0, pl.ds(col0, 16)]
            scat = scatbuf[c, 0, pl.ds(col0, 16)]
            addr = iota16 * N + jnp.maximum(scat, 0)
            plsc.store_scatter(s_chunk, [addr], w_vec, mask=scat >= 0)
        pltpu.sync_copy(s_chunk,
                        s_flat_out.at[pl.ds((tile_base + ci * CH) * N, CH * N)])
        for c in range(K5):
            scat = scatbuf[c, 0, pl.ds(col0, 16)]
            addr = iota16 * N + jnp.maximum(scat, 0)
            plsc.store_scatter(s_chunk, [addr], zf16, mask=scat >= 0)
        return carry

    lax.fori_loop(0, NCH, chunk_body, 0)


def _k4(w3, scat3):
    mesh = plsc.VectorSubcoreMesh(core_axis_name="c", subcore_axis_name="s")
    return pl.kernel(
        _k4_body,
        out_type=jax.ShapeDtypeStruct((ROWS * N,), jnp.float32),
        mesh=mesh,
        compiler_params=pltpu.CompilerParams(needs_layout_passes=False),
        scratch_types=[
            pltpu.VMEM((W48, 1, RPT), jnp.float32),
            pltpu.VMEM((W48, 1, RPT), jnp.int32),
            pltpu.VMEM((CH * N,), jnp.float32),
        ],
    )(w3, scat3)


def _k5_body(v_ref, s_ref, o_ref):
    o_ref[0] = lax.dot_general(v_ref[0], s_ref[0], (((1,), (1,)), ((), ())),
                               preferred_element_type=jnp.float32)


def _k5(v_r, s_dense):
    return pl.pallas_call(
        _k5_body,
        grid=(B, N // BM),
        in_specs=[
            pl.BlockSpec((1, CV, N), lambda b, i: (b, 0, 0)),
            pl.BlockSpec((1, BM, N), lambda b, i: (b, i, 0)),
        ],
        out_specs=pl.BlockSpec((1, CV, BM), lambda b, i: (b, 0, i)),
        out_shape=jax.ShapeDtypeStruct((B, CV, N), jnp.float32),
        compiler_params=pltpu.CompilerParams(
            dimension_semantics=("parallel", "parallel")),
    )(v_r, s_dense)


@jax.jit
def kernel(q, k, q_prune, k_prune, v, prev_attn_top_k_idx):
    q_r = q.reshape(B, C, N)
    k_r = k.reshape(B, C, N)
    qp_r = q_prune.reshape(B, CP, N)
    kp_r = k_prune.reshape(B, CP, N)
    v_r = v.reshape(B, CV, N)

    a_d = _k1(q_r, k_r, qp_r, kp_r)

    idx3, s3 = _k2(prev_attn_top_k_idx.reshape(ROWS * NK),
                   a_d.reshape(ROWS * N))

    idx_t = idx3.reshape(W48, ROWS)
    w_t, scat_t, topk_t = _k3(s3.reshape(W48, ROWS), idx_t)

    s_flat = _k4(w_t.reshape(W48, NTILES, RPT),
                 scat_t.reshape(W48, NTILES, RPT))

    out_cn = _k5(v_r, s_flat.reshape(B, N, N))

    output = out_cn.reshape(B, CV, H, W)
    this_attn_top_k_idx = topk_t.T.reshape(B, N, NK)
    conf = w_t[W48 - 1].reshape(B, 1, H, W)
    return output, this_attn_top_k_idx, conf
